# baseline (speedup 1.0000x reference)
import functools

import jax
import jax.numpy as jnp
from jax import lax
from jax.experimental import pallas as pl
from jax.experimental.pallas import tpu as pltpu
from jax.experimental.pallas import tpu_sc as plsc

N_NODES = 10000
N_EDGES = 320000
D = 128

_NC = 2
_NS = 16
_NW = _NC * _NS
_SLICE_M = (7, 5, 4, 3, 2, 2, 1, 1)
_UNIT = 12800
_CH = 80
_NB = 5


def _proj_body(x_ref, wa_ref, wb_ref, p_ref, q_ref):
    xv = x_ref[...]
    p_ref[...] = jnp.dot(xv, wa_ref[...], preferred_element_type=jnp.float32)
    q_ref[...] = jnp.dot(xv, wb_ref[...], preferred_element_type=jnp.float32)


def _project_nodes(x, wa, wb):
    blk = 2000
    grid = N_NODES // blk
    return pl.pallas_call(
        _proj_body,
        grid=(grid,),
        in_specs=[
            pl.BlockSpec((blk, D), lambda i: (i, 0)),
            pl.BlockSpec((D, D), lambda i: (0, 0)),
            pl.BlockSpec((D, D), lambda i: (0, 0)),
        ],
        out_specs=[
            pl.BlockSpec((blk, D), lambda i: (i, 0)),
            pl.BlockSpec((blk, D), lambda i: (i, 0)),
        ],
        out_shape=[
            jax.ShapeDtypeStruct((N_NODES, D), jnp.float32),
            jax.ShapeDtypeStruct((N_NODES, D), jnp.float32),
        ],
    )(x, wa, wb)


def _gather_body(slice_base, epw, nchunk, p_hbm, q_hbm, src_hbm, dst_hbm, g_hbm,
                 si_all, di_all, rs0, rs1, rs2, rs3, rs4,
                 rd0, rd1, rd2, rd3, rd4,
                 gsem_s, gsem_d, wsem):
    wid = lax.axis_index("s") * _NC + lax.axis_index("c")
    wbase = wid * epw
    rs = [rs0, rs1, rs2, rs3, rs4]
    rd = [rd0, rd1, rd2, rd3, rd4]

    def gather_descs(k, b):
        isl = pl.ds(k * _CH, _CH)
        return (
            pltpu.make_async_copy(p_hbm.at[si_all.at[isl]], rs[b], gsem_s.at[b]),
            pltpu.make_async_copy(q_hbm.at[di_all.at[isl]], rd[b], gsem_d.at[b]),
        )

    def wb_desc(k, b):
        osl = pl.ds(wbase + k * _CH, _CH)
        return pltpu.make_async_copy(rs[b], g_hbm.at[osl], wsem.at[b])

    pltpu.sync_copy(src_hbm.at[pl.ds(slice_base + wbase, epw)], si_all)
    pltpu.sync_copy(dst_hbm.at[pl.ds(slice_base + wbase, epw)], di_all)
    for dsc in gather_descs(0, 0):
        dsc.start()

    def outer(kk, carry):
        for b in range(_NB):
            k = kk * _NB + b
            k1 = k + 1
            nb = (b + 1) % _NB

            @pl.when(k1 < nchunk)
            def _prefetch():
                @pl.when(k1 >= _NB)
                def _reclaim():
                    wb_desc(k1 - _NB, nb).wait()

                for dsc in gather_descs(k1, nb):
                    dsc.start()

            for dsc in gather_descs(k, b):
                dsc.wait()

            def row_body(r, c2):
                for j in range(D // 16):
                    sl = pl.ds(j * 16, 16)
                    rs[b][r, sl] = rs[b][r, sl] + rd[b][r, sl]
                return c2

            lax.fori_loop(0, _CH, row_body, 0)
            wb_desc(k, b).start()
        return carry

    lax.fori_loop(0, nchunk // _NB, outer, 0)
    for b in range(_NB):
        wb_desc(nchunk - _NB + b, b).wait()


def _gather_add(p, q, src, dst, e_off, es):
    epw = es // _NW
    nchunk = epw // _CH
    mesh = plsc.VectorSubcoreMesh(core_axis_name="c", subcore_axis_name="s")
    fn = functools.partial(
        pl.kernel,
        mesh=mesh,
        out_type=jax.ShapeDtypeStruct((es, D), jnp.float32),
        scratch_types=(
            [pltpu.VMEM((epw,), jnp.int32)] * 2
            + [pltpu.VMEM((_CH, D), jnp.float32)] * (2 * _NB)
            + [pltpu.SemaphoreType.DMA((_NB,))] * 3
        ),
    )(functools.partial(_gather_body, e_off, epw, nchunk))
    return fn(p, q, src, dst)


def _mlp_body(g_ref, eat_ref, w1c_ref, b1_ref, w2_ref, b2_ref, ot_ref):
    c = lax.dot_general(eat_ref[...], w1c_ref[...],
                        (((0,), (0,)), ((), ())),
                        preferred_element_type=jnp.float32)
    h = g_ref[...] + c + b1_ref[...]
    h = jnp.maximum(h, 0.0)
    ot_ref[...] = lax.dot_general(w2_ref[...], h,
                                  (((0,), (1,)), ((), ())),
                                  preferred_element_type=jnp.float32
                                  ) + b2_ref[...]


def _edge_mlp(g, ea_t, w1c, b1, w2, b2, e_off):
    blk = 2560
    grid = g.shape[0] // blk
    off = e_off // blk
    de = ea_t.shape[0]
    eo = w2.shape[1]
    out_t = pl.pallas_call(
        _mlp_body,
        grid=(grid,),
        in_specs=[
            pl.BlockSpec((blk, D), lambda i: (i, 0)),
            pl.BlockSpec((de, blk), lambda i: (0, off + i)),
            pl.BlockSpec((de, D), lambda i: (0, 0)),
            pl.BlockSpec((1, D), lambda i: (0, 0)),
            pl.BlockSpec((D, eo), lambda i: (0, 0)),
            pl.BlockSpec((eo, 1), lambda i: (0, 0)),
        ],
        out_specs=pl.BlockSpec((eo, blk), lambda i: (0, i)),
        out_shape=jax.ShapeDtypeStruct((eo, g.shape[0]), jnp.float32),
    )(g, ea_t, w1c, b1, w2, b2)
    return out_t


def kernel(x, edge_index, edge_attr, W1, b1, W2, b2):
    wa = W1[:D]
    wb = W1[D:2 * D]
    w1c = W1[2 * D:]
    p, q = _project_nodes(x, wa, wb)
    ea_t = edge_attr.T
    src = edge_index[0]
    dst = edge_index[1]
    b1r = b1.reshape(1, D)
    b2r = b2.reshape(-1, 1)
    outs = []
    e_off = 0
    for m in _SLICE_M:
        es = m * _UNIT
        g = _gather_add(p, q, src, dst, e_off, es)
        outs.append(_edge_mlp(g, ea_t, w1c, b1r, W2, b2r, e_off))
        e_off += es
    return jnp.concatenate(outs, axis=1).T


# speedup vs baseline: 1.0297x; 1.0297x over previous
import functools

import jax
import jax.numpy as jnp
from jax import lax
from jax.experimental import pallas as pl
from jax.experimental.pallas import tpu as pltpu
from jax.experimental.pallas import tpu_sc as plsc

N_NODES = 10000
N_EDGES = 320000
D = 128

_NC = 2
_NS = 16
_NW = _NC * _NS
_SLICE_M = (7, 6, 5, 4, 3)
_UNIT = 12800
_CH = 80
_NB = 5


def _proj_body(x_ref, wa_ref, wb_ref, p_ref, q_ref):
    xv = x_ref[...]
    p_ref[...] = jnp.dot(xv, wa_ref[...], preferred_element_type=jnp.float32)
    q_ref[...] = jnp.dot(xv, wb_ref[...], preferred_element_type=jnp.float32)


def _project_nodes(x, wa, wb):
    blk = 2000
    grid = N_NODES // blk
    return pl.pallas_call(
        _proj_body,
        grid=(grid,),
        in_specs=[
            pl.BlockSpec((blk, D), lambda i: (i, 0)),
            pl.BlockSpec((D, D), lambda i: (0, 0)),
            pl.BlockSpec((D, D), lambda i: (0, 0)),
        ],
        out_specs=[
            pl.BlockSpec((blk, D), lambda i: (i, 0)),
            pl.BlockSpec((blk, D), lambda i: (i, 0)),
        ],
        out_shape=[
            jax.ShapeDtypeStruct((N_NODES, D), jnp.float32),
            jax.ShapeDtypeStruct((N_NODES, D), jnp.float32),
        ],
    )(x, wa, wb)


def _gather_body(slice_base, epw, nchunk, p_hbm, q_hbm, src_hbm, dst_hbm, g_hbm,
                 si_all, di_all, rs0, rs1, rs2, rs3, rs4,
                 rd0, rd1, rd2, rd3, rd4,
                 gsem_s, gsem_d, wsem):
    wid = lax.axis_index("s") * _NC + lax.axis_index("c")
    wbase = wid * epw
    rs = [rs0, rs1, rs2, rs3, rs4]
    rd = [rd0, rd1, rd2, rd3, rd4]

    def gather_descs(k, b):
        isl = pl.ds(k * _CH, _CH)
        return (
            pltpu.make_async_copy(p_hbm.at[si_all.at[isl]], rs[b], gsem_s.at[b]),
            pltpu.make_async_copy(q_hbm.at[di_all.at[isl]], rd[b], gsem_d.at[b]),
        )

    def wb_desc(k, b):
        osl = pl.ds(wbase + k * _CH, _CH)
        return pltpu.make_async_copy(rs[b], g_hbm.at[osl], wsem.at[b])

    pltpu.sync_copy(src_hbm.at[pl.ds(slice_base + wbase, epw)], si_all)
    pltpu.sync_copy(dst_hbm.at[pl.ds(slice_base + wbase, epw)], di_all)
    for dsc in gather_descs(0, 0):
        dsc.start()

    def outer(kk, carry):
        for b in range(_NB):
            k = kk * _NB + b
            k1 = k + 1
            nb = (b + 1) % _NB

            @pl.when(k1 < nchunk)
            def _prefetch():
                @pl.when(k1 >= _NB)
                def _reclaim():
                    wb_desc(k1 - _NB, nb).wait()

                for dsc in gather_descs(k1, nb):
                    dsc.start()

            for dsc in gather_descs(k, b):
                dsc.wait()

            def row_body(r, c2):
                for j in range(D // 16):
                    sl = pl.ds(j * 16, 16)
                    rs[b][r, sl] = rs[b][r, sl] + rd[b][r, sl]
                return c2

            lax.fori_loop(0, _CH, row_body, 0)
            wb_desc(k, b).start()
        return carry

    lax.fori_loop(0, nchunk // _NB, outer, 0)
    for b in range(_NB):
        wb_desc(nchunk - _NB + b, b).wait()


def _gather_add(p, q, src, dst, e_off, es):
    epw = es // _NW
    nchunk = epw // _CH
    mesh = plsc.VectorSubcoreMesh(core_axis_name="c", subcore_axis_name="s")
    fn = functools.partial(
        pl.kernel,
        mesh=mesh,
        out_type=jax.ShapeDtypeStruct((es, D), jnp.float32),
        scratch_types=(
            [pltpu.VMEM((epw,), jnp.int32)] * 2
            + [pltpu.VMEM((_CH, D), jnp.float32)] * (2 * _NB)
            + [pltpu.SemaphoreType.DMA((_NB,))] * 3
        ),
    )(functools.partial(_gather_body, e_off, epw, nchunk))
    return fn(p, q, src, dst)


def _mlp_body(g_ref, eat_ref, w1c_ref, b1_ref, w2_ref, b2_ref, ot_ref):
    c = lax.dot_general(eat_ref[...], w1c_ref[...],
                        (((0,), (0,)), ((), ())),
                        preferred_element_type=jnp.float32)
    h = g_ref[...] + c + b1_ref[...]
    h = jnp.maximum(h, 0.0)
    ot_ref[...] = lax.dot_general(w2_ref[...], h,
                                  (((0,), (1,)), ((), ())),
                                  preferred_element_type=jnp.float32
                                  ) + b2_ref[...]


def _edge_mlp(g, ea_t, w1c, b1, w2, b2, e_off):
    blk = 2560
    grid = g.shape[0] // blk
    off = e_off // blk
    de = ea_t.shape[0]
    eo = w2.shape[1]
    out_t = pl.pallas_call(
        _mlp_body,
        grid=(grid,),
        in_specs=[
            pl.BlockSpec((blk, D), lambda i: (i, 0)),
            pl.BlockSpec((de, blk), lambda i: (0, off + i)),
            pl.BlockSpec((de, D), lambda i: (0, 0)),
            pl.BlockSpec((1, D), lambda i: (0, 0)),
            pl.BlockSpec((D, eo), lambda i: (0, 0)),
            pl.BlockSpec((eo, 1), lambda i: (0, 0)),
        ],
        out_specs=pl.BlockSpec((eo, blk), lambda i: (0, i)),
        out_shape=jax.ShapeDtypeStruct((eo, g.shape[0]), jnp.float32),
    )(g, ea_t, w1c, b1, w2, b2)
    return out_t


def kernel(x, edge_index, edge_attr, W1, b1, W2, b2):
    wa = W1[:D]
    wb = W1[D:2 * D]
    w1c = W1[2 * D:]
    p, q = _project_nodes(x, wa, wb)
    ea_t = edge_attr.T
    src = edge_index[0]
    dst = edge_index[1]
    b1r = b1.reshape(1, D)
    b2r = b2.reshape(-1, 1)
    outs = []
    e_off = 0
    for m in _SLICE_M:
        es = m * _UNIT
        g = _gather_add(p, q, src, dst, e_off, es)
        outs.append(_edge_mlp(g, ea_t, w1c, b1r, W2, b2r, e_off))
        e_off += es
    return jnp.concatenate(outs, axis=1).T


# speedup vs baseline: 1.0344x; 1.0045x over previous
import functools

import jax
import jax.numpy as jnp
from jax import lax
from jax.experimental import pallas as pl
from jax.experimental.pallas import tpu as pltpu
from jax.experimental.pallas import tpu_sc as plsc

N_NODES = 10000
N_EDGES = 320000
D = 128

_NC = 2
_NS = 16
_NW = _NC * _NS
_SLICE_M = (8, 6, 5, 4, 2)
_UNIT = 12800
_CH = 80
_NB = 5


def _proj_body(x_ref, wa_ref, wb_ref, p_ref, q_ref):
    xv = x_ref[...]
    p_ref[...] = jnp.dot(xv, wa_ref[...], preferred_element_type=jnp.float32)
    q_ref[...] = jnp.dot(xv, wb_ref[...], preferred_element_type=jnp.float32)


def _project_nodes(x, wa, wb):
    blk = 2000
    grid = N_NODES // blk
    return pl.pallas_call(
        _proj_body,
        grid=(grid,),
        in_specs=[
            pl.BlockSpec((blk, D), lambda i: (i, 0)),
            pl.BlockSpec((D, D), lambda i: (0, 0)),
            pl.BlockSpec((D, D), lambda i: (0, 0)),
        ],
        out_specs=[
            pl.BlockSpec((blk, D), lambda i: (i, 0)),
            pl.BlockSpec((blk, D), lambda i: (i, 0)),
        ],
        out_shape=[
            jax.ShapeDtypeStruct((N_NODES, D), jnp.float32),
            jax.ShapeDtypeStruct((N_NODES, D), jnp.float32),
        ],
    )(x, wa, wb)


def _gather_body(slice_base, epw, nchunk, p_hbm, q_hbm, src_hbm, dst_hbm, g_hbm,
                 si_all, di_all, rs0, rs1, rs2, rs3, rs4,
                 rd0, rd1, rd2, rd3, rd4,
                 gsem_s, gsem_d, wsem):
    wid = lax.axis_index("s") * _NC + lax.axis_index("c")
    wbase = wid * epw
    rs = [rs0, rs1, rs2, rs3, rs4]
    rd = [rd0, rd1, rd2, rd3, rd4]

    def gather_descs(k, b):
        isl = pl.ds(k * _CH, _CH)
        return (
            pltpu.make_async_copy(p_hbm.at[si_all.at[isl]], rs[b], gsem_s.at[b]),
            pltpu.make_async_copy(q_hbm.at[di_all.at[isl]], rd[b], gsem_d.at[b]),
        )

    def wb_desc(k, b):
        osl = pl.ds(wbase + k * _CH, _CH)
        return pltpu.make_async_copy(rs[b], g_hbm.at[osl], wsem.at[b])

    pltpu.sync_copy(src_hbm.at[pl.ds(slice_base + wbase, epw)], si_all)
    pltpu.sync_copy(dst_hbm.at[pl.ds(slice_base + wbase, epw)], di_all)
    for dsc in gather_descs(0, 0):
        dsc.start()

    def outer(kk, carry):
        for b in range(_NB):
            k = kk * _NB + b
            k1 = k + 1
            nb = (b + 1) % _NB

            @pl.when(k1 < nchunk)
            def _prefetch():
                @pl.when(k1 >= _NB)
                def _reclaim():
                    wb_desc(k1 - _NB, nb).wait()

                for dsc in gather_descs(k1, nb):
                    dsc.start()

            for dsc in gather_descs(k, b):
                dsc.wait()

            def row_body(r, c2):
                for j in range(D // 16):
                    sl = pl.ds(j * 16, 16)
                    rs[b][r, sl] = rs[b][r, sl] + rd[b][r, sl]
                return c2

            lax.fori_loop(0, _CH, row_body, 0)
            wb_desc(k, b).start()
        return carry

    lax.fori_loop(0, nchunk // _NB, outer, 0)
    for b in range(_NB):
        wb_desc(nchunk - _NB + b, b).wait()


def _gather_add(p, q, src, dst, e_off, es):
    epw = es // _NW
    nchunk = epw // _CH
    mesh = plsc.VectorSubcoreMesh(core_axis_name="c", subcore_axis_name="s")
    fn = functools.partial(
        pl.kernel,
        mesh=mesh,
        out_type=jax.ShapeDtypeStruct((es, D), jnp.float32),
        scratch_types=(
            [pltpu.VMEM((epw,), jnp.int32)] * 2
            + [pltpu.VMEM((_CH, D), jnp.float32)] * (2 * _NB)
            + [pltpu.SemaphoreType.DMA((_NB,))] * 3
        ),
    )(functools.partial(_gather_body, e_off, epw, nchunk))
    return fn(p, q, src, dst)


def _mlp_body(g_ref, eat_ref, w1c_ref, b1_ref, w2_ref, b2_ref, ot_ref):
    c = lax.dot_general(eat_ref[...], w1c_ref[...],
                        (((0,), (0,)), ((), ())),
                        preferred_element_type=jnp.float32)
    h = g_ref[...] + c + b1_ref[...]
    h = jnp.maximum(h, 0.0)
    ot_ref[...] = lax.dot_general(w2_ref[...], h,
                                  (((0,), (1,)), ((), ())),
                                  preferred_element_type=jnp.float32
                                  ) + b2_ref[...]


def _edge_mlp(g, ea_t, w1c, b1, w2, b2, e_off):
    blk = 2560
    grid = g.shape[0] // blk
    off = e_off // blk
    de = ea_t.shape[0]
    eo = w2.shape[1]
    out_t = pl.pallas_call(
        _mlp_body,
        grid=(grid,),
        in_specs=[
            pl.BlockSpec((blk, D), lambda i: (i, 0)),
            pl.BlockSpec((de, blk), lambda i: (0, off + i)),
            pl.BlockSpec((de, D), lambda i: (0, 0)),
            pl.BlockSpec((1, D), lambda i: (0, 0)),
            pl.BlockSpec((D, eo), lambda i: (0, 0)),
            pl.BlockSpec((eo, 1), lambda i: (0, 0)),
        ],
        out_specs=pl.BlockSpec((eo, blk), lambda i: (0, i)),
        out_shape=jax.ShapeDtypeStruct((eo, g.shape[0]), jnp.float32),
    )(g, ea_t, w1c, b1, w2, b2)
    return out_t


def kernel(x, edge_index, edge_attr, W1, b1, W2, b2):
    wa = W1[:D]
    wb = W1[D:2 * D]
    w1c = W1[2 * D:]
    p, q = _project_nodes(x, wa, wb)
    ea_t = edge_attr.T
    src = edge_index[0]
    dst = edge_index[1]
    b1r = b1.reshape(1, D)
    b2r = b2.reshape(-1, 1)
    outs = []
    e_off = 0
    for m in _SLICE_M:
        es = m * _UNIT
        g = _gather_add(p, q, src, dst, e_off, es)
        outs.append(_edge_mlp(g, ea_t, w1c, b1r, W2, b2r, e_off))
        e_off += es
    return jnp.concatenate(outs, axis=1).T
